# speedup vs baseline: 1.4714x; 1.4714x over previous
import functools

import jax
import jax.numpy as jnp
from jax import lax
from jax.experimental import pallas as pl
from jax.experimental.pallas import tpu as pltpu
from jax.experimental.pallas import tpu_sc as plsc

_VOCAB = 100000
_EMB = 128
_HID = 768
_MAXPOS = 4096
_TYPES = 4
_B, _S = 4, 4096
_TOK = _B * _S
_EPS = 1e-12


_info = plsc.get_sparse_core_info()
_NC, _NS = _info.num_cores, _info.num_subcores
_NW = _NC * _NS
_ROWS_PER_W = _TOK // _NW
_CHUNK = 128
_NCHUNK = _ROWS_PER_W // _CHUNK


def _sc_gather_body(table_hbm, idx_hbm, out_hbm, idx_v, rows_v, sem):
    wid = lax.axis_index("s") * _NC + lax.axis_index("c")
    pltpu.sync_copy(idx_hbm.at[pl.ds(wid * _NCHUNK, _NCHUNK)], idx_v)
    copies = []
    for j in range(_NCHUNK):
        copies.append(
            pltpu.async_copy(
                table_hbm.at[idx_v.at[j]],
                rows_v.at[pl.ds(j * _CHUNK, _CHUNK)],
                sem,
            )
        )
    for cp in copies:
        cp.wait()
    pltpu.sync_copy(rows_v, out_hbm.at[pl.ds(wid * _ROWS_PER_W, _ROWS_PER_W)])


_sc_gather = functools.partial(
    pl.kernel,
    mesh=plsc.VectorSubcoreMesh(core_axis_name="c", subcore_axis_name="s"),
    out_type=jax.ShapeDtypeStruct((_TOK, _EMB), jnp.float32),
    scratch_types=[
        pltpu.VMEM((_NCHUNK, _CHUNK), jnp.int32),
        pltpu.VMEM((_ROWS_PER_W, _EMB), jnp.float32),
        pltpu.SemaphoreType.DMA,
    ],
)(_sc_gather_body)


_BT = 512
_NB = _TOK // _BT


def _tc_body(g_ref, p_ref, t_ref, te_ref, gam_ref, bet_ref, w_ref, b_ref, o_ref):
    acc = g_ref[...] + p_ref[...]
    tid = t_ref[...]
    te = te_ref[...]
    for k in range(_TYPES):
        acc = acc + jnp.where(tid == k, te[k:k + 1, :], 0.0)
    mu = jnp.mean(acc, axis=1, keepdims=True)
    d = acc - mu
    var = jnp.mean(d * d, axis=1, keepdims=True)
    y = d * lax.rsqrt(var + _EPS) * gam_ref[...] + bet_ref[...]
    o_ref[...] = (
        jnp.dot(y, w_ref[...], preferred_element_type=jnp.float32) + b_ref[...]
    )


def _tc_fused(gathered, pos_emb, type_ids_col, te_pad, gamma2, beta2, W, b2):
    return pl.pallas_call(
        _tc_body,
        grid=(_NB,),
        in_specs=[
            pl.BlockSpec((_BT, _EMB), lambda i: (i, 0)),
            pl.BlockSpec((_BT, _EMB), lambda i: (i % (_MAXPOS // _BT), 0)),
            pl.BlockSpec((_BT, 1), lambda i: (i, 0)),
            pl.BlockSpec((8, _EMB), lambda i: (0, 0)),
            pl.BlockSpec((1, _EMB), lambda i: (0, 0)),
            pl.BlockSpec((1, _EMB), lambda i: (0, 0)),
            pl.BlockSpec((_EMB, _HID), lambda i: (0, 0)),
            pl.BlockSpec((1, _HID), lambda i: (0, 0)),
        ],
        out_specs=pl.BlockSpec((_BT, _HID), lambda i: (i, 0)),
        out_shape=jax.ShapeDtypeStruct((_TOK, _HID), jnp.float32),
    )(gathered, pos_emb, type_ids_col, te_pad, gamma2, beta2, W, b2)


def kernel(input_ids, type_ids, word_emb, pos_emb, type_emb, gamma, beta, W, b):
    ids2d = input_ids.astype(jnp.int32).reshape(_NW * _NCHUNK, _CHUNK)
    gathered = _sc_gather(word_emb, ids2d)
    te_pad = jnp.zeros((8, _EMB), jnp.float32).at[:_TYPES].set(type_emb)
    out = _tc_fused(
        gathered,
        pos_emb,
        type_ids.astype(jnp.int32).reshape(_TOK, 1),
        te_pad,
        gamma.reshape(1, _EMB),
        beta.reshape(1, _EMB),
        W,
        b.reshape(1, _HID),
    )
    return out.reshape(_B, _S, _HID)


# speedup vs baseline: 1.6298x; 1.1076x over previous
import functools

import jax
import jax.numpy as jnp
from jax import lax
from jax.experimental import pallas as pl
from jax.experimental.pallas import tpu as pltpu
from jax.experimental.pallas import tpu_sc as plsc

_VOCAB = 100000
_EMB = 128
_HID = 768
_MAXPOS = 4096
_TYPES = 4
_B, _S = 4, 4096
_TOK = _B * _S
_EPS = 1e-12


_NC, _NS = 2, 16
_NW = _NC * _NS
_ROWS_PER_W = _TOK // _NW
_CHUNK = 128
_NCHUNK = _ROWS_PER_W // _CHUNK


def _sc_gather_body(table_hbm, idx_hbm, out_hbm, idx_v, rows_v, sem):
    wid = lax.axis_index("s") * _NC + lax.axis_index("c")
    pltpu.sync_copy(idx_hbm.at[pl.ds(wid * _NCHUNK, _NCHUNK)], idx_v)
    copies = []
    for j in range(_NCHUNK):
        copies.append(
            pltpu.async_copy(
                table_hbm.at[idx_v.at[j]],
                rows_v.at[pl.ds(j * _CHUNK, _CHUNK)],
                sem,
            )
        )
    for cp in copies:
        cp.wait()
    pltpu.sync_copy(rows_v, out_hbm.at[pl.ds(wid * _ROWS_PER_W, _ROWS_PER_W)])


@functools.cache
def _sc_gather():
    return functools.partial(
        pl.kernel,
        mesh=plsc.VectorSubcoreMesh(core_axis_name="c", subcore_axis_name="s"),
        out_type=jax.ShapeDtypeStruct((_TOK, _EMB), jnp.float32),
        scratch_types=[
            pltpu.VMEM((_NCHUNK, _CHUNK), jnp.int32),
            pltpu.VMEM((_ROWS_PER_W, _EMB), jnp.float32),
            pltpu.SemaphoreType.DMA,
        ],
    )(_sc_gather_body)


_BT = 512
_JB = _MAXPOS // _BT


def _tc_body(g_ref, p_ref, t_ref, te_ref, gam_ref, bet_ref, w_ref, b_ref, o_ref):
    tid = t_ref[...]
    oh = (
        lax.broadcasted_iota(jnp.int32, (8, _BT), 0) == tid
    ).astype(jnp.float32)
    te = lax.dot_general(
        oh, te_ref[...], (((0,), (0,)), ((), ())),
        preferred_element_type=jnp.float32,
    )
    acc = g_ref[...] + p_ref[...] + te
    mu = jnp.mean(acc, axis=1, keepdims=True)
    d = acc - mu
    var = jnp.mean(d * d, axis=1, keepdims=True)
    y = d * lax.rsqrt(var + _EPS) * gam_ref[...] + bet_ref[...]
    o_ref[...] = (
        jnp.dot(y, w_ref[...], preferred_element_type=jnp.float32) + b_ref[...]
    )


def _tc_fused(gathered, pos_emb, type_ids_row, te_pad, gamma2, beta2, W, b2):
    return pl.pallas_call(
        _tc_body,
        grid=(_JB, _B),
        in_specs=[
            pl.BlockSpec((_BT, _EMB), lambda j, bi: (bi * _JB + j, 0)),
            pl.BlockSpec((_BT, _EMB), lambda j, bi: (j, 0)),
            pl.BlockSpec((1, _BT), lambda j, bi: (0, bi * _JB + j)),
            pl.BlockSpec((8, _EMB), lambda j, bi: (0, 0)),
            pl.BlockSpec((1, _EMB), lambda j, bi: (0, 0)),
            pl.BlockSpec((1, _EMB), lambda j, bi: (0, 0)),
            pl.BlockSpec((_EMB, _HID), lambda j, bi: (0, 0)),
            pl.BlockSpec((1, _HID), lambda j, bi: (0, 0)),
        ],
        out_specs=pl.BlockSpec((_BT, _HID), lambda j, bi: (bi * _JB + j, 0)),
        out_shape=jax.ShapeDtypeStruct((_TOK, _HID), jnp.float32),
    )(gathered, pos_emb, type_ids_row, te_pad, gamma2, beta2, W, b2)


def kernel(input_ids, type_ids, word_emb, pos_emb, type_emb, gamma, beta, W, b):
    ids2d = input_ids.astype(jnp.int32).reshape(_NW * _NCHUNK, _CHUNK)
    gathered = _sc_gather()(word_emb, ids2d)
    te_pad = jnp.zeros((8, _EMB), jnp.float32).at[:_TYPES].set(type_emb)
    out = _tc_fused(
        gathered,
        pos_emb,
        type_ids.astype(jnp.int32).reshape(1, _TOK),
        te_pad,
        gamma.reshape(1, _EMB),
        beta.reshape(1, _EMB),
        W,
        b.reshape(1, _HID),
    )
    return out.reshape(_B, _S, _HID)


# speedup vs baseline: 1.9545x; 1.1993x over previous
import functools

import jax
import jax.numpy as jnp
from jax import lax
from jax.experimental import pallas as pl
from jax.experimental.pallas import tpu as pltpu
from jax.experimental.pallas import tpu_sc as plsc

_VOCAB = 100000
_EMB = 128
_HID = 768
_MAXPOS = 4096
_TYPES = 4
_B, _S = 4, 4096
_TOK = _B * _S
_EPS = 1e-12


_NC, _NS = 2, 16
_NW = _NC * _NS
_ROWS_PER_W = _TOK // _NW
_CHUNK = 128
_NCHUNK = _ROWS_PER_W // _CHUNK


def _sc_gather_body(table_hbm, idx_hbm, out_hbm, idx_v, rows_v, gsem, wsem):
    wid = lax.axis_index("s") * _NC + lax.axis_index("c")
    pltpu.sync_copy(idx_hbm.at[pl.ds(wid * _NCHUNK, _NCHUNK)], idx_v)
    gathers = []
    for j in range(_NCHUNK):
        gathers.append(
            pltpu.async_copy(
                table_hbm.at[idx_v.at[j]],
                rows_v.at[pl.ds(j * _CHUNK, _CHUNK)],
                gsem,
            )
        )
    writes = []
    for j in range(_NCHUNK):
        gathers[j].wait()
        writes.append(
            pltpu.async_copy(
                rows_v.at[pl.ds(j * _CHUNK, _CHUNK)],
                out_hbm.at[pl.ds(wid * _ROWS_PER_W + j * _CHUNK, _CHUNK)],
                wsem,
            )
        )
    for cp in writes:
        cp.wait()


@functools.cache
def _sc_gather():
    return functools.partial(
        pl.kernel,
        mesh=plsc.VectorSubcoreMesh(core_axis_name="c", subcore_axis_name="s"),
        out_type=jax.ShapeDtypeStruct((_TOK, _EMB), jnp.float32),
        scratch_types=[
            pltpu.VMEM((_NCHUNK, _CHUNK), jnp.int32),
            pltpu.VMEM((_ROWS_PER_W, _EMB), jnp.float32),
            pltpu.SemaphoreType.DMA,
            pltpu.SemaphoreType.DMA,
        ],
    )(_sc_gather_body)


_BT = 1024
_JB = _MAXPOS // _BT


def _tc_body(g_ref, p_ref, t_ref, te_ref, gam_ref, bet_ref, w_ref, b_ref, o_ref):
    tid = t_ref[...]
    oh = (
        lax.broadcasted_iota(jnp.int32, (8, _BT), 0) == tid
    ).astype(jnp.float32)
    te = lax.dot_general(
        oh, te_ref[...], (((0,), (0,)), ((), ())),
        preferred_element_type=jnp.float32,
    )
    acc = g_ref[...] + p_ref[...] + te
    mu = jnp.mean(acc, axis=1, keepdims=True)
    d = acc - mu
    var = jnp.mean(d * d, axis=1, keepdims=True)
    y = d * lax.rsqrt(var + _EPS) * gam_ref[...] + bet_ref[...]
    o_ref[...] = (
        jnp.dot(y, w_ref[...], preferred_element_type=jnp.float32) + b_ref[...]
    )


def _tc_fused(gathered, pos_emb, type_ids_row, te_pad, gamma2, beta2, W, b2):
    return pl.pallas_call(
        _tc_body,
        grid=(_JB, _B),
        in_specs=[
            pl.BlockSpec((_BT, _EMB), lambda j, bi: (bi * _JB + j, 0)),
            pl.BlockSpec((_BT, _EMB), lambda j, bi: (j, 0)),
            pl.BlockSpec((1, _BT), lambda j, bi: (0, bi * _JB + j)),
            pl.BlockSpec((8, _EMB), lambda j, bi: (0, 0)),
            pl.BlockSpec((1, _EMB), lambda j, bi: (0, 0)),
            pl.BlockSpec((1, _EMB), lambda j, bi: (0, 0)),
            pl.BlockSpec((_EMB, _HID), lambda j, bi: (0, 0)),
            pl.BlockSpec((1, _HID), lambda j, bi: (0, 0)),
        ],
        out_specs=pl.BlockSpec((_BT, _HID), lambda j, bi: (bi * _JB + j, 0)),
        out_shape=jax.ShapeDtypeStruct((_TOK, _HID), jnp.float32),
    )(gathered, pos_emb, type_ids_row, te_pad, gamma2, beta2, W, b2)


def kernel(input_ids, type_ids, word_emb, pos_emb, type_emb, gamma, beta, W, b):
    ids2d = input_ids.astype(jnp.int32).reshape(_NW * _NCHUNK, _CHUNK)
    gathered = _sc_gather()(word_emb, ids2d)
    te_pad = jnp.zeros((8, _EMB), jnp.float32).at[:_TYPES].set(type_emb)
    out = _tc_fused(
        gathered,
        pos_emb,
        type_ids.astype(jnp.int32).reshape(1, _TOK),
        te_pad,
        gamma.reshape(1, _EMB),
        beta.reshape(1, _EMB),
        W,
        b.reshape(1, _HID),
    )
    return out.reshape(_B, _S, _HID)
